# baseline (speedup 1.0000x reference)
import functools

import jax
import jax.numpy as jnp
from jax import lax
from jax.experimental import pallas as pl
from jax.experimental.pallas import tpu as pltpu
from jax.experimental.pallas import tpu_sc as plsc

N_ROWS = 100000
N_CLS = 80
TR = N_ROWS // 8

NW = 32
SC_TR = 5120
PER_W = SC_TR // NW
K = 16
NCH = PER_W // K
RPT = PER_W * 8
CHW = K * 8 * N_CLS

TC_BLK = 4000
TC_OFF = 10
TC_NB = 15
TC_OVL = SC_TR * 8 - TC_OFF * TC_BLK

LN2 = 0.6931471805599453
PC0 = 6.944574454190702e-05
PC1 = 0.9962619482337948
PC2 = -0.4664424386275713
PC3 = 0.21866548366222927
PC4 = -0.05545931374208457

_sc_mesh = plsc.VectorSubcoreMesh(core_axis_name="c", subcore_axis_name="s")


def _ln16(x):
    bits = lax.bitcast_convert_type(x, jnp.int32)
    e = (bits >> 23) - 127
    m = lax.bitcast_convert_type((bits & 0x7FFFFF) | 0x3F800000, jnp.float32)
    t = m - 1.0
    p = PC4 * t + PC3
    p = p * t + PC2
    p = p * t + PC1
    p = p * t + PC0
    return e.astype(jnp.float32) * LN2 + p


@functools.partial(
    pl.kernel,
    mesh=_sc_mesh,
    out_type=jax.ShapeDtypeStruct((NW, 16), jnp.float32),
    scratch_types=[
        pltpu.VMEM((K * 8, N_CLS), jnp.float32),
        pltpu.VMEM((CHW,), jnp.float32),
        pltpu.VMEM((3, RPT), jnp.float32),
        pltpu.VMEM((K * 8,), jnp.int32),
        pltpu.VMEM((K * 8,), jnp.float32),
        pltpu.VMEM_SHARED((16 * CHW,), jnp.float32),
        pltpu.VMEM((16,), jnp.float32),
        pltpu.SemaphoreType.DMA,
        pltpu.SemaphoreType.DMA,
        pltpu.SemaphoreType.DMA,
    ],
    cost_estimate=pl.CostEstimate(
        flops=200_000_000, transcendentals=0, bytes_accessed=21_000_000
    ),
    compiler_params=pltpu.CompilerParams(use_tc_tiling_on_sc=True),
)
def _sc_part(pred_hbm, aux_hbm, out_hbm, buf, flat, auxv, idxb, patb,
             shared, accv, sem, sema, semg):
    cid = lax.axis_index("c")
    sid = lax.axis_index("s")
    wid = sid * 2 + cid
    base = wid * PER_W

    pltpu.async_copy(aux_hbm.at[:, pl.ds(base * 8, RPT)], auxv, sema).wait()

    iot = lax.iota(jnp.int32, 16)

    def chunk(ch, accs):
        acc_d, acc_c = accs
        pltpu.async_copy(
            pred_hbm.at[pl.ds((base + ch * K) * 8, K * 8)], buf, sem
        ).wait()

        def tr(r, a):
            t = a
            for v in range(5):
                p = buf[r, v * 16:(v + 1) * 16]
                flat[pl.ds(r * N_CLS + v * 16, 16)] = p
                l1p = _ln16(1.0 - p)
                t = t + l1p * (p * p)
            return t

        acc_d = lax.fori_loop(0, K * 8, tr, acc_d)

        pltpu.sync_copy(flat, shared.at[pl.ds(sid * CHW, CHW)])

        def idxg(g, carry):
            off = ch * (K * 8) + g * 16
            lab = auxv[0, pl.ds(off, 16)]
            li = jnp.clip(lab, 0.0, float(N_CLS - 1)).astype(jnp.int32)
            r16 = g * 16 + iot
            idxb[pl.ds(g * 16, 16)] = sid * CHW + r16 * N_CLS + li
            return carry

        lax.fori_loop(0, K * 8 // 16, idxg, 0)
        pltpu.async_copy(shared.at[idxb], patb, semg).wait()

        def grp(g, a):
            off = ch * (K * 8) + g * 16
            lab = auxv[0, pl.ds(off, 16)]
            s = auxv[1, pl.ds(off, 16)]
            w = auxv[2, pl.ds(off, 16)]
            valid = (lab >= 0.0) & (lab < float(N_CLS))
            p_at = patb[pl.ds(g * 16, 16)]
            lp = _ln16(p_at)
            l1p = _ln16(1.0 - p_at)
            posval = -((lp - l1p) * s + l1p) * w
            negat = l1p * (p_at * p_at) * -0.75
            return a + jnp.where(valid, posval - negat, 0.0)

        acc_c = lax.fori_loop(0, K * 8 // 16, grp, acc_c)
        return (acc_d, acc_c)

    z = jnp.zeros((16,), jnp.float32)
    acc_d, acc_c = lax.fori_loop(0, NCH, chunk, (z, z))
    accv[...] = acc_d * -0.75 + acc_c
    pltpu.sync_copy(accv, out_hbm.at[wid])


def _tc_body(pred_ref, aux_ref, out_ref, acc_ref):
    i = pl.program_id(0)
    nb = pl.num_programs(0)
    blk = TC_BLK

    p = pred_ref[...]
    labf = aux_ref[0, 0:1, :]
    s = aux_ref[0, 1:2, :]
    w = aux_ref[0, 2:3, :]
    slab = jnp.where((labf >= 0.0) & (labf < N_CLS), labf, -1.0)

    logp = jnp.maximum(jnp.log(p), -100.0)
    log1mp = jnp.maximum(jnp.log(1.0 - p), -100.0)
    neg = log1mp * (p * p) * -0.75

    jota = jax.lax.broadcasted_iota(jnp.int32, (N_CLS, blk), 0).astype(jnp.float32)
    match = jota == slab
    zero = jnp.zeros((), jnp.float32)
    e_ws = jnp.where(match, -(w * s), zero)
    e_w1 = jnp.where(match, -(w * (1.0 - s)), zero)
    e_1 = jnp.where(match, -1.0, zero)

    m = (
        jnp.dot(e_ws, logp, preferred_element_type=jnp.float32)
        + jnp.dot(e_w1, log1mp, preferred_element_type=jnp.float32)
        + jnp.dot(e_1, neg, preferred_element_type=jnp.float32)
    )
    diag = jax.lax.broadcasted_iota(jnp.int32, (N_CLS, N_CLS), 0) == (
        jax.lax.broadcasted_iota(jnp.int32, (N_CLS, N_CLS), 1)
    )
    corr8 = jnp.where(diag, m, zero).reshape(N_CLS // 8, 8, N_CLS).sum(axis=0)

    @pl.when(i == 0)
    def _init():
        sufx = neg[TC_OVL:].reshape((blk - TC_OVL) // 8, 8, N_CLS).sum(axis=0)
        acc_ref[...] = sufx + corr8

    @pl.when(i > 0)
    def _acc():
        full = neg.reshape(blk // 8, 8, N_CLS).sum(axis=0)
        acc_ref[...] += full + corr8

    @pl.when(i == nb - 1)
    def _fin():
        out_ref[0, 0] = jnp.sum(acc_ref[...])


def kernel(pred, label, score, weight):
    n_rows, n_cls = pred.shape

    aux = jnp.stack([label.astype(jnp.float32), score, weight])

    r0 = TC_OFF * TC_BLK
    labt = jnp.concatenate(
        [jnp.full((TC_OVL,), -1.0, jnp.float32),
         label[r0 + TC_OVL:].astype(jnp.float32)]
    )
    aux_tc = (
        jnp.stack([labt, score[r0:], weight[r0:]])
        .reshape(3, TC_NB, TC_BLK)
        .transpose(1, 0, 2)
    )

    tc_out = pl.pallas_call(
        _tc_body,
        grid=(TC_NB,),
        in_specs=[
            pl.BlockSpec((TC_BLK, n_cls), lambda i: (i + TC_OFF, 0)),
            pl.BlockSpec((1, 3, TC_BLK), lambda i: (i, 0, 0)),
        ],
        out_specs=pl.BlockSpec(
            (1, 1), lambda i: (0, 0), memory_space=pltpu.SMEM
        ),
        out_shape=jax.ShapeDtypeStruct((1, 1), jnp.float32),
        scratch_shapes=[pltpu.VMEM((8, n_cls), jnp.float32)],
        cost_estimate=pl.CostEstimate(
            flops=300_000_000, transcendentals=10_000_000,
            bytes_accessed=31_000_000,
        ),
    )(pred, aux_tc)

    sc_out = _sc_part(pred, aux)
    return (tc_out[0, 0] + jnp.sum(sc_out)) * (1.0 / n_rows)


# speedup vs baseline: 1.3425x; 1.3425x over previous
import functools

import jax
import jax.numpy as jnp
from jax.experimental import pallas as pl
from jax.experimental.pallas import tpu as pltpu


def _body(pred_ref, aux_ref, out_ref, acc_ref, *, n_rows, n_cls, blk):
    i = pl.program_id(0)
    nb = pl.num_programs(0)

    p = pred_ref[...]
    labf = aux_ref[0, 0:1, :]
    s = aux_ref[0, 1:2, :]
    w = aux_ref[0, 2:3, :]
    slab = jnp.where((labf >= 0.0) & (labf < n_cls), labf, -1.0)

    logp = jnp.maximum(jnp.log(p), -100.0)
    log1mp = jnp.maximum(jnp.log(1.0 - p), -100.0)
    neg = log1mp * (p * p) * -0.75

    jota = jax.lax.broadcasted_iota(jnp.int32, (n_cls, blk), 0).astype(jnp.float32)
    match = jota == slab
    zero = jnp.zeros((), jnp.float32)
    e_ws = jnp.where(match, -(w * s), zero)
    e_w1 = jnp.where(match, -(w * (1.0 - s)), zero)
    e_1 = jnp.where(match, -1.0, zero)

    m = (
        jnp.dot(e_ws, logp, preferred_element_type=jnp.float32)
        + jnp.dot(e_w1, log1mp, preferred_element_type=jnp.float32)
        + jnp.dot(e_1, neg, preferred_element_type=jnp.float32)
    )
    diag = jax.lax.broadcasted_iota(jnp.int32, (n_cls, n_cls), 0) == (
        jax.lax.broadcasted_iota(jnp.int32, (n_cls, n_cls), 1)
    )
    corr = jnp.where(diag, m, zero)

    part = (
        neg.reshape(blk // 8, 8, n_cls).sum(axis=0)
        + corr.reshape(n_cls // 8, 8, n_cls).sum(axis=0)
    )

    @pl.when(i == 0)
    def _init():
        acc_ref[...] = part

    @pl.when(i > 0)
    def _acc():
        acc_ref[...] += part

    @pl.when(i == nb - 1)
    def _fin():
        out_ref[0, 0] = jnp.sum(acc_ref[...]) * (1.0 / n_rows)


def kernel(pred, label, score, weight):
    n_rows, n_cls = pred.shape
    blk = 10000
    nb = n_rows // blk

    aux = jnp.stack([label.astype(jnp.float32), score, weight])
    aux = aux.reshape(3, nb, blk).transpose(1, 0, 2)

    out = pl.pallas_call(
        functools.partial(_body, n_rows=n_rows, n_cls=n_cls, blk=blk),
        grid=(nb,),
        in_specs=[
            pl.BlockSpec((blk, n_cls), lambda i: (i, 0)),
            pl.BlockSpec((1, 3, blk), lambda i: (i, 0, 0)),
        ],
        out_specs=pl.BlockSpec(
            (1, 1), lambda i: (0, 0), memory_space=pltpu.SMEM
        ),
        out_shape=jax.ShapeDtypeStruct((1, 1), jnp.float32),
        scratch_shapes=[pltpu.VMEM((8, n_cls), jnp.float32)],
    )(pred, aux)
    return out[0, 0]
